# baseline (speedup 1.0000x reference)
import functools

import jax
import jax.numpy as jnp
from jax import lax
from jax.experimental import pallas as pl
from jax.experimental.pallas import tpu as pltpu
from jax.experimental.pallas import tpu_sc as plsc

NC = 2
NS = 16
LANES = 128
CH = 8


def _sc_deg_kernel(n_nodes, n_rows):
    rw = n_rows // (NC * NS)
    mesh = plsc.VectorSubcoreMesh(core_axis_name="c", subcore_axis_name="s")

    @functools.partial(
        pl.kernel,
        out_type=[
            jax.ShapeDtypeStruct((n_rows, LANES), jnp.float32),
            jax.ShapeDtypeStruct((NC, n_nodes), jnp.float32),
        ],
        mesh=mesh,
        compiler_params=pltpu.CompilerParams(use_tc_tiling_on_sc=False),
        scratch_types=[
            pltpu.VMEM((3, CH, LANES), jnp.float32),
            pltpu.VMEM((3, CH, LANES), jnp.float32),
            pltpu.VMEM((3, CH, LANES), jnp.float32),
            pltpu.VMEM((3, CH, LANES), jnp.int32),
            pltpu.VMEM((16,), jnp.float32),
            pltpu.VMEM((16,), jnp.float32),
            pltpu.VMEM_SHARED((n_nodes,), jnp.float32),
            pltpu.SemaphoreType.DMA,
            pltpu.SemaphoreType.DMA,
            pltpu.SemaphoreType.DMA,
        ],
    )
    def k(a01_hbm, rc_hbm, ws0_hbm, ws1_hbm, zn_hbm,
          w_hbm, degp_hbm,
          a0_v, a1_v, w_v, cidx_v, ws0_v, ws1_v, deg_sh, isem, ssem, wsem):
        c = lax.axis_index("c")
        s = lax.axis_index("s")
        wid = s * NC + c

        @pl.when(s == 0)
        def _():
            pltpu.sync_copy(zn_hbm, deg_sh)

        pltpu.sync_copy(ws0_hbm, ws0_v)
        pltpu.sync_copy(ws1_hbm, ws1_v)
        plsc.subcore_barrier()
        ws0 = ws0_v[...]
        ws1 = ws1_v[...]

        nchunk = rw // CH

        def base_of(ci):
            return pl.multiple_of(wid * rw + ci * CH, CH)

        def fire_idx(ci):
            sl = ci % 3
            return [
                pltpu.async_copy(a01_hbm.at[0].at[pl.ds(base_of(ci), CH)],
                                 a0_v.at[sl], isem),
                pltpu.async_copy(a01_hbm.at[1].at[pl.ds(base_of(ci), CH)],
                                 a1_v.at[sl], isem),
                pltpu.async_copy(rc_hbm.at[1].at[pl.ds(base_of(ci), CH)],
                                 cidx_v.at[sl], isem),
            ]

        pend_idx = {0: fire_idx(0)}
        pend_sc = {}
        pend_w = {}
        for ci in range(nchunk):
            sl = ci % 3
            if ci >= 2:
                for hh in pend_sc.pop(ci - 2):
                    hh.wait()
                pend_w.pop(ci - 2).wait()
            for hh in pend_idx.pop(ci):
                hh.wait()
            if ci + 1 < nchunk:
                pend_idx[ci + 1] = fire_idx(ci + 1)
            def wcomp(t, carry2):
                j = t >> 3
                qs = pl.ds((t & (LANES // 16 - 1)) * 16, 16)
                w_v[sl, j, qs] = (ws0 * a0_v[sl, j, qs]
                                  + ws1 * a1_v[sl, j, qs])
                return carry2
            lax.fori_loop(0, CH * (LANES // 16), wcomp, 0)
            pend_w[ci] = pltpu.async_copy(
                w_v.at[sl], w_hbm.at[pl.ds(base_of(ci), CH)], wsem)
            pend_sc[ci] = [
                pltpu.async_copy(w_v.at[sl].at[j],
                                 deg_sh.at[cidx_v.at[sl].at[j]],
                                 ssem, add=True)
                for j in range(CH)
            ]
        for ci in sorted(pend_sc):
            for hh in pend_sc[ci]:
                hh.wait()
            pend_w.pop(ci).wait()
        plsc.subcore_barrier()

        @pl.when(s == 0)
        def _():
            pltpu.sync_copy(deg_sh, degp_hbm.at[c])

    return k


def _sc_msg_kernel(n_nodes, n_rows, d):
    rw = n_rows // (NC * NS)
    mesh = plsc.VectorSubcoreMesh(core_axis_name="c", subcore_axis_name="s")
    rows_per_tile = (n_nodes // (8 * NS)) * 8
    rem_start = rows_per_tile * NS
    rem = n_nodes - rem_start

    @functools.partial(
        pl.kernel,
        out_type=jax.ShapeDtypeStruct((NC, n_nodes, d), jnp.float32),
        mesh=mesh,
        compiler_params=pltpu.CompilerParams(use_tc_tiling_on_sc=False),
        scratch_types=[
            pltpu.VMEM((4, CH, LANES), jnp.int32),
            pltpu.VMEM((4, CH, LANES), jnp.int32),
            pltpu.VMEM((4, CH, LANES), jnp.float32),
            pltpu.VMEM((2, CH, LANES, d), jnp.float32),
            pltpu.VMEM_SHARED((n_nodes, d), jnp.float32),
            pltpu.SemaphoreType.DMA,
            pltpu.SemaphoreType.DMA,
            pltpu.SemaphoreType.DMA,
        ],
    )
    def k(xs_hbm, rc_hbm, w_hbm, znd_hbm,
          accp_hbm,
          ridx_v, cidx_v, w_v, rows_v, acc_sh, isem, gsem, ssem):
        c = lax.axis_index("c")
        s = lax.axis_index("s")
        wid = s * NC + c

        tb = pl.multiple_of(s * rows_per_tile, 8)
        pltpu.sync_copy(znd_hbm.at[pl.ds(tb, rows_per_tile)],
                        acc_sh.at[pl.ds(tb, rows_per_tile)])
        if rem:
            @pl.when(s == 0)
            def _():
                pltpu.sync_copy(znd_hbm.at[pl.ds(rem_start, rem)],
                                acc_sh.at[pl.ds(rem_start, rem)])
        plsc.subcore_barrier()

        nchunk = rw // CH

        def base_of(ci):
            return pl.multiple_of(wid * rw + ci * CH, CH)

        def fire_idx(ci):
            sl = ci % 4
            return [
                pltpu.async_copy(rc_hbm.at[0].at[pl.ds(base_of(ci), CH)],
                                 ridx_v.at[sl], isem),
                pltpu.async_copy(rc_hbm.at[1].at[pl.ds(base_of(ci), CH)],
                                 cidx_v.at[sl], isem),
                pltpu.async_copy(w_hbm.at[pl.ds(base_of(ci), CH)],
                                 w_v.at[sl], isem),
            ]

        def fire_gathers(ci):
            return [
                pltpu.async_copy(xs_hbm.at[ridx_v.at[ci % 4].at[j]],
                                 rows_v.at[ci % 2].at[j], gsem)
                for j in range(CH)
            ]

        pend_idx = {0: fire_idx(0)}
        for hh in pend_idx.pop(0):
            hh.wait()
        pend_g = {0: fire_gathers(0)}
        pend_idx[1] = fire_idx(1)
        pend_sc = {}
        for ci in range(nchunk):
            sl = ci % 4
            rb = ci % 2
            if ci >= 1:
                for hh in pend_sc.pop(ci - 1):
                    hh.wait()
            if ci + 1 < nchunk:
                for hh in pend_idx.pop(ci + 1):
                    hh.wait()
                pend_g[ci + 1] = fire_gathers(ci + 1)
                if ci + 2 < nchunk:
                    pend_idx[ci + 2] = fire_idx(ci + 2)
            for hh in pend_g.pop(ci):
                hh.wait()

            def scale(t, carry2):
                j = t >> 3
                g = t & (LANES // 16 - 1)
                w16 = w_v[sl, j, pl.ds(g * 16, 16)]
                for l in range(16):
                    wv = w16[l]
                    ii = g * 16 + l
                    for q in range(d // 16):
                        qs = pl.ds(q * 16, 16)
                        rows_v[rb, j, ii, qs] = rows_v[rb, j, ii, qs] * wv
                return carry2
            lax.fori_loop(0, CH * (LANES // 16), scale, 0)

            pend_sc[ci] = [
                pltpu.async_copy(rows_v.at[rb].at[j],
                                 acc_sh.at[cidx_v.at[sl].at[j]],
                                 ssem, add=True)
                for j in range(CH)
            ]
        for ci in sorted(pend_sc):
            for hh in pend_sc[ci]:
                hh.wait()
        plsc.subcore_barrier()

        @pl.when(s == 0)
        def _():
            pltpu.sync_copy(acc_sh, accp_hbm.at[c])

    return k


def _tc_norm_matmul(n_nodes, f_in, h, blk):
    def body(p0_ref, p1_ref, x_ref, w1_ref, xw_ref, xs_ref, dis_ref, inv_ref):
        deg = 1.0 + p0_ref[...] + p1_ref[...]
        dis = lax.rsqrt(deg)
        dis_ref[...] = dis
        inv_ref[...] = 1.0 / deg
        xw = jnp.dot(x_ref[...], w1_ref[...], preferred_element_type=jnp.float32)
        xw_ref[...] = xw
        xs_ref[...] = xw * dis

    return pl.pallas_call(
        body,
        grid=(n_nodes // blk,),
        in_specs=[
            pl.BlockSpec((blk, 1), lambda i: (i, 0)),
            pl.BlockSpec((blk, 1), lambda i: (i, 0)),
            pl.BlockSpec((blk, f_in), lambda i: (i, 0)),
            pl.BlockSpec((f_in, h), lambda i: (0, 0)),
        ],
        out_specs=[
            pl.BlockSpec((blk, h), lambda i: (i, 0)),
            pl.BlockSpec((blk, h), lambda i: (i, 0)),
            pl.BlockSpec((blk, 1), lambda i: (i, 0)),
            pl.BlockSpec((blk, 1), lambda i: (i, 0)),
        ],
        out_shape=[
            jax.ShapeDtypeStruct((n_nodes, h), jnp.float32),
            jax.ShapeDtypeStruct((n_nodes, h), jnp.float32),
            jax.ShapeDtypeStruct((n_nodes, 1), jnp.float32),
            jax.ShapeDtypeStruct((n_nodes, 1), jnp.float32),
        ],
    )


def _tc_layer1_finish(n_nodes, h, c_out, blk):
    def body(a0_ref, a1_ref, xw1_ref, dis_ref, inv_ref, b1_ref, w2_ref,
             xw2_ref, xs2_ref):
        dis = dis_ref[...]
        act = dis * (a0_ref[0] + a1_ref[0]) + inv_ref[...] * xw1_ref[...]
        act = jnp.maximum(act + b1_ref[...], 0.0)
        xw2 = jnp.dot(act, w2_ref[...], preferred_element_type=jnp.float32)
        xw2_ref[...] = xw2
        xs2_ref[...] = xw2 * dis

    return pl.pallas_call(
        body,
        grid=(n_nodes // blk,),
        in_specs=[
            pl.BlockSpec((1, blk, h), lambda i: (0, i, 0)),
            pl.BlockSpec((1, blk, h), lambda i: (1, i, 0)),
            pl.BlockSpec((blk, h), lambda i: (i, 0)),
            pl.BlockSpec((blk, 1), lambda i: (i, 0)),
            pl.BlockSpec((blk, 1), lambda i: (i, 0)),
            pl.BlockSpec((1, h), lambda i: (0, 0)),
            pl.BlockSpec((h, c_out), lambda i: (0, 0)),
        ],
        out_specs=[
            pl.BlockSpec((blk, c_out), lambda i: (i, 0)),
            pl.BlockSpec((blk, c_out), lambda i: (i, 0)),
        ],
        out_shape=[
            jax.ShapeDtypeStruct((n_nodes, c_out), jnp.float32),
            jax.ShapeDtypeStruct((n_nodes, c_out), jnp.float32),
        ],
    )


def _tc_layer2_finish(n_nodes, c_out, blk):
    def body(a0_ref, a1_ref, xw2_ref, dis_ref, inv_ref, b2_ref, out_ref):
        out_ref[...] = (dis_ref[...] * (a0_ref[0] + a1_ref[0])
                        + inv_ref[...] * xw2_ref[...] + b2_ref[...])

    return pl.pallas_call(
        body,
        grid=(n_nodes // blk,),
        in_specs=[
            pl.BlockSpec((1, blk, c_out), lambda i: (0, i, 0)),
            pl.BlockSpec((1, blk, c_out), lambda i: (1, i, 0)),
            pl.BlockSpec((blk, c_out), lambda i: (i, 0)),
            pl.BlockSpec((blk, 1), lambda i: (i, 0)),
            pl.BlockSpec((blk, 1), lambda i: (i, 0)),
            pl.BlockSpec((1, c_out), lambda i: (0, 0)),
        ],
        out_specs=pl.BlockSpec((blk, c_out), lambda i: (i, 0)),
        out_shape=jax.ShapeDtypeStruct((n_nodes, c_out), jnp.float32),
    )


def kernel(x, edge_index, A0, A1, ws, W1, b1, W2, b2):
    n_nodes, f_in = x.shape
    n_edges = A0.shape[0]
    h = W1.shape[1]
    c_out = W2.shape[1]

    group = NC * NS * CH * LANES
    n_rows = ((n_edges + group - 1) // group) * (group // LANES)
    e_pad = n_rows * LANES
    pad = e_pad - n_edges

    iot = jnp.arange(pad, dtype=jnp.int32) % n_nodes
    rc2d = jnp.concatenate(
        [edge_index[0], iot, edge_index[1], iot]).reshape(2, n_rows, LANES)
    zpad = jnp.zeros((pad,), jnp.float32)
    a01_2d = jnp.concatenate([A0, zpad, A1, zpad]).reshape(2, n_rows, LANES)
    ws0s = jnp.full((16,), ws[0], jnp.float32)
    ws1s = jnp.full((16,), ws[1], jnp.float32)
    zn = jnp.zeros((n_nodes,), jnp.float32)
    zh = jnp.zeros((n_nodes, h), jnp.float32)
    zc = jnp.zeros((n_nodes, c_out), jnp.float32)

    blk = 1000
    w2d, degp = _sc_deg_kernel(n_nodes, n_rows)(a01_2d, rc2d, ws0s, ws1s, zn)

    xw1, xs1, dis, inv = _tc_norm_matmul(n_nodes, f_in, h, blk)(
        degp[0].reshape(n_nodes, 1), degp[1].reshape(n_nodes, 1), x, W1)

    acc1 = _sc_msg_kernel(n_nodes, n_rows, h)(xs1, rc2d, w2d, zh)

    xw2, xs2 = _tc_layer1_finish(n_nodes, h, c_out, blk)(
        acc1, acc1, xw1, dis, inv, b1.reshape(1, h), W2)

    acc2 = _sc_msg_kernel(n_nodes, n_rows, c_out)(xs2, rc2d, w2d, zc)

    out = _tc_layer2_finish(n_nodes, c_out, blk)(
        acc2, acc2, xw2, dis, inv, b2.reshape(1, c_out))
    return out


# speedup vs baseline: 1.0120x; 1.0120x over previous
import functools

import jax
import jax.numpy as jnp
from jax import lax
from jax.experimental import pallas as pl
from jax.experimental.pallas import tpu as pltpu
from jax.experimental.pallas import tpu_sc as plsc

NC = 2
NS = 16
LANES = 128
CH = 8


def _sc_deg_kernel(n_nodes, n_rows):
    rw = n_rows // (NC * NS)
    mesh = plsc.VectorSubcoreMesh(core_axis_name="c", subcore_axis_name="s")

    @functools.partial(
        pl.kernel,
        out_type=[
            jax.ShapeDtypeStruct((n_rows, LANES), jnp.float32),
            jax.ShapeDtypeStruct((NC, n_nodes), jnp.float32),
        ],
        mesh=mesh,
        compiler_params=pltpu.CompilerParams(use_tc_tiling_on_sc=False),
        scratch_types=[
            pltpu.VMEM((3, CH, LANES), jnp.float32),
            pltpu.VMEM((3, CH, LANES), jnp.float32),
            pltpu.VMEM((3, CH, LANES), jnp.float32),
            pltpu.VMEM((3, CH, LANES), jnp.int32),
            pltpu.VMEM((16,), jnp.float32),
            pltpu.VMEM((16,), jnp.float32),
            pltpu.VMEM_SHARED((n_nodes,), jnp.float32),
            pltpu.SemaphoreType.DMA,
            pltpu.SemaphoreType.DMA,
            pltpu.SemaphoreType.DMA,
        ],
    )
    def k(a01_hbm, rc_hbm, ws0_hbm, ws1_hbm, zn_hbm,
          w_hbm, degp_hbm,
          a0_v, a1_v, w_v, cidx_v, ws0_v, ws1_v, deg_sh, isem, ssem, wsem):
        c = lax.axis_index("c")
        s = lax.axis_index("s")
        wid = s * NC + c

        @pl.when(s == 0)
        def _():
            pltpu.sync_copy(zn_hbm, deg_sh)

        pltpu.sync_copy(ws0_hbm, ws0_v)
        pltpu.sync_copy(ws1_hbm, ws1_v)
        plsc.subcore_barrier()
        ws0 = ws0_v[...]
        ws1 = ws1_v[...]

        nchunk = rw // CH

        def base_of(ci):
            return pl.multiple_of(wid * rw + ci * CH, CH)

        def fire_idx(ci):
            sl = ci % 3
            return [
                pltpu.async_copy(a01_hbm.at[0].at[pl.ds(base_of(ci), CH)],
                                 a0_v.at[sl], isem),
                pltpu.async_copy(a01_hbm.at[1].at[pl.ds(base_of(ci), CH)],
                                 a1_v.at[sl], isem),
                pltpu.async_copy(rc_hbm.at[1].at[pl.ds(base_of(ci), CH)],
                                 cidx_v.at[sl], isem),
            ]

        pend_idx = {0: fire_idx(0)}
        pend_sc = {}
        pend_w = {}
        for ci in range(nchunk):
            sl = ci % 3
            if ci >= 2:
                for hh in pend_sc.pop(ci - 2):
                    hh.wait()
                pend_w.pop(ci - 2).wait()
            for hh in pend_idx.pop(ci):
                hh.wait()
            if ci + 1 < nchunk:
                pend_idx[ci + 1] = fire_idx(ci + 1)
            def wcomp(t, carry2):
                j = t >> 3
                qs = pl.ds((t & (LANES // 16 - 1)) * 16, 16)
                w_v[sl, j, qs] = (ws0 * a0_v[sl, j, qs]
                                  + ws1 * a1_v[sl, j, qs])
                return carry2
            lax.fori_loop(0, CH * (LANES // 16), wcomp, 0)
            pend_w[ci] = pltpu.async_copy(
                w_v.at[sl], w_hbm.at[pl.ds(base_of(ci), CH)], wsem)
            pend_sc[ci] = [
                pltpu.async_copy(w_v.at[sl].at[j],
                                 deg_sh.at[cidx_v.at[sl].at[j]],
                                 ssem, add=True)
                for j in range(CH)
            ]
        for ci in sorted(pend_sc):
            for hh in pend_sc[ci]:
                hh.wait()
            pend_w.pop(ci).wait()
        plsc.subcore_barrier()

        @pl.when(s == 0)
        def _():
            pltpu.sync_copy(deg_sh, degp_hbm.at[c])

    return k


def _sc_msg_kernel(n_nodes, n_rows, d):
    rw = n_rows // (NC * NS)
    mesh = plsc.VectorSubcoreMesh(core_axis_name="c", subcore_axis_name="s")
    rows_per_tile = (n_nodes // (8 * NS)) * 8
    rem_start = rows_per_tile * NS
    rem = n_nodes - rem_start

    @functools.partial(
        pl.kernel,
        out_type=jax.ShapeDtypeStruct((NC, n_nodes, d), jnp.float32),
        mesh=mesh,
        compiler_params=pltpu.CompilerParams(use_tc_tiling_on_sc=False),
        scratch_types=[
            pltpu.VMEM((4, CH, LANES), jnp.int32),
            pltpu.VMEM((4, CH, LANES), jnp.int32),
            pltpu.VMEM((4, CH, LANES), jnp.float32),
            pltpu.VMEM((2, CH, LANES, d), jnp.float32),
            pltpu.VMEM_SHARED((n_nodes, d), jnp.float32),
            pltpu.VMEM_SHARED((n_nodes, d), jnp.float32),
            pltpu.SemaphoreType.DMA,
            pltpu.SemaphoreType.DMA,
            pltpu.SemaphoreType.DMA,
        ],
    )
    def k(xs_hbm, rc_hbm, w_hbm, znd_hbm,
          accp_hbm,
          ridx_v, cidx_v, w_v, rows_v, xs_sh, acc_sh, isem, gsem, ssem):
        c = lax.axis_index("c")
        s = lax.axis_index("s")
        wid = s * NC + c

        tb = pl.multiple_of(s * rows_per_tile, 8)
        pltpu.sync_copy(xs_hbm.at[pl.ds(tb, rows_per_tile)],
                        xs_sh.at[pl.ds(tb, rows_per_tile)])
        pltpu.sync_copy(znd_hbm.at[pl.ds(tb, rows_per_tile)],
                        acc_sh.at[pl.ds(tb, rows_per_tile)])
        if rem:
            @pl.when(s == 0)
            def _():
                pltpu.sync_copy(xs_hbm.at[pl.ds(rem_start, rem)],
                                xs_sh.at[pl.ds(rem_start, rem)])
                pltpu.sync_copy(znd_hbm.at[pl.ds(rem_start, rem)],
                                acc_sh.at[pl.ds(rem_start, rem)])
        plsc.subcore_barrier()

        nchunk = rw // CH

        def base_of(ci):
            return pl.multiple_of(wid * rw + ci * CH, CH)

        def fire_idx(ci):
            sl = ci % 4
            return [
                pltpu.async_copy(rc_hbm.at[0].at[pl.ds(base_of(ci), CH)],
                                 ridx_v.at[sl], isem),
                pltpu.async_copy(rc_hbm.at[1].at[pl.ds(base_of(ci), CH)],
                                 cidx_v.at[sl], isem),
                pltpu.async_copy(w_hbm.at[pl.ds(base_of(ci), CH)],
                                 w_v.at[sl], isem),
            ]

        def fire_gathers(ci):
            return [
                pltpu.async_copy(xs_sh.at[ridx_v.at[ci % 4].at[j]],
                                 rows_v.at[ci % 2].at[j], gsem)
                for j in range(CH)
            ]

        pend_idx = {0: fire_idx(0)}
        for hh in pend_idx.pop(0):
            hh.wait()
        pend_g = {0: fire_gathers(0)}
        pend_idx[1] = fire_idx(1)
        pend_sc = {}
        for ci in range(nchunk):
            sl = ci % 4
            rb = ci % 2
            if ci >= 1:
                for hh in pend_sc.pop(ci - 1):
                    hh.wait()
            if ci + 1 < nchunk:
                for hh in pend_idx.pop(ci + 1):
                    hh.wait()
                pend_g[ci + 1] = fire_gathers(ci + 1)
                if ci + 2 < nchunk:
                    pend_idx[ci + 2] = fire_idx(ci + 2)
            for hh in pend_g.pop(ci):
                hh.wait()

            def scale(t, carry2):
                j = t >> 3
                g = t & (LANES // 16 - 1)
                w16 = w_v[sl, j, pl.ds(g * 16, 16)]
                for l in range(16):
                    wv = w16[l]
                    ii = g * 16 + l
                    for q in range(d // 16):
                        qs = pl.ds(q * 16, 16)
                        rows_v[rb, j, ii, qs] = rows_v[rb, j, ii, qs] * wv
                return carry2
            lax.fori_loop(0, CH * (LANES // 16), scale, 0)

            pend_sc[ci] = [
                pltpu.async_copy(rows_v.at[rb].at[j],
                                 acc_sh.at[cidx_v.at[sl].at[j]],
                                 ssem, add=True)
                for j in range(CH)
            ]
        for ci in sorted(pend_sc):
            for hh in pend_sc[ci]:
                hh.wait()
        plsc.subcore_barrier()

        @pl.when(s == 0)
        def _():
            pltpu.sync_copy(acc_sh, accp_hbm.at[c])

    return k


def _tc_matmul1(n_nodes, f_in, h, blk):
    def body(x_ref, w1_ref, xw_ref):
        xw_ref[...] = jnp.dot(x_ref[...], w1_ref[...],
                              preferred_element_type=jnp.float32)

    return pl.pallas_call(
        body,
        grid=(n_nodes // blk,),
        in_specs=[
            pl.BlockSpec((blk, f_in), lambda i: (i, 0)),
            pl.BlockSpec((f_in, h), lambda i: (0, 0)),
        ],
        out_specs=pl.BlockSpec((blk, h), lambda i: (i, 0)),
        out_shape=jax.ShapeDtypeStruct((n_nodes, h), jnp.float32),
    )


def _tc_norm(n_nodes, h, blk):
    def body(p0_ref, p1_ref, xw_ref, xs_ref, dis_ref, inv_ref):
        deg = 1.0 + p0_ref[...] + p1_ref[...]
        dis = lax.rsqrt(deg)
        dis_ref[...] = dis
        inv_ref[...] = 1.0 / deg
        xs_ref[...] = xw_ref[...] * dis

    return pl.pallas_call(
        body,
        grid=(n_nodes // blk,),
        in_specs=[
            pl.BlockSpec((blk, 1), lambda i: (i, 0)),
            pl.BlockSpec((blk, 1), lambda i: (i, 0)),
            pl.BlockSpec((blk, h), lambda i: (i, 0)),
        ],
        out_specs=[
            pl.BlockSpec((blk, h), lambda i: (i, 0)),
            pl.BlockSpec((blk, 1), lambda i: (i, 0)),
            pl.BlockSpec((blk, 1), lambda i: (i, 0)),
        ],
        out_shape=[
            jax.ShapeDtypeStruct((n_nodes, h), jnp.float32),
            jax.ShapeDtypeStruct((n_nodes, 1), jnp.float32),
            jax.ShapeDtypeStruct((n_nodes, 1), jnp.float32),
        ],
    )


def _tc_layer1_finish(n_nodes, h, c_out, blk):
    def body(a0_ref, a1_ref, xw1_ref, dis_ref, inv_ref, b1_ref, w2_ref,
             xw2_ref, xs2_ref):
        dis = dis_ref[...]
        act = dis * (a0_ref[0] + a1_ref[0]) + inv_ref[...] * xw1_ref[...]
        act = jnp.maximum(act + b1_ref[...], 0.0)
        xw2 = jnp.dot(act, w2_ref[...], preferred_element_type=jnp.float32)
        xw2_ref[...] = xw2
        xs2_ref[...] = xw2 * dis

    return pl.pallas_call(
        body,
        grid=(n_nodes // blk,),
        in_specs=[
            pl.BlockSpec((1, blk, h), lambda i: (0, i, 0)),
            pl.BlockSpec((1, blk, h), lambda i: (1, i, 0)),
            pl.BlockSpec((blk, h), lambda i: (i, 0)),
            pl.BlockSpec((blk, 1), lambda i: (i, 0)),
            pl.BlockSpec((blk, 1), lambda i: (i, 0)),
            pl.BlockSpec((1, h), lambda i: (0, 0)),
            pl.BlockSpec((h, c_out), lambda i: (0, 0)),
        ],
        out_specs=[
            pl.BlockSpec((blk, c_out), lambda i: (i, 0)),
            pl.BlockSpec((blk, c_out), lambda i: (i, 0)),
        ],
        out_shape=[
            jax.ShapeDtypeStruct((n_nodes, c_out), jnp.float32),
            jax.ShapeDtypeStruct((n_nodes, c_out), jnp.float32),
        ],
    )


def _tc_layer2_finish(n_nodes, c_out, blk):
    def body(a0_ref, a1_ref, xw2_ref, dis_ref, inv_ref, b2_ref, out_ref):
        out_ref[...] = (dis_ref[...] * (a0_ref[0] + a1_ref[0])
                        + inv_ref[...] * xw2_ref[...] + b2_ref[...])

    return pl.pallas_call(
        body,
        grid=(n_nodes // blk,),
        in_specs=[
            pl.BlockSpec((1, blk, c_out), lambda i: (0, i, 0)),
            pl.BlockSpec((1, blk, c_out), lambda i: (1, i, 0)),
            pl.BlockSpec((blk, c_out), lambda i: (i, 0)),
            pl.BlockSpec((blk, 1), lambda i: (i, 0)),
            pl.BlockSpec((blk, 1), lambda i: (i, 0)),
            pl.BlockSpec((1, c_out), lambda i: (0, 0)),
        ],
        out_specs=pl.BlockSpec((blk, c_out), lambda i: (i, 0)),
        out_shape=jax.ShapeDtypeStruct((n_nodes, c_out), jnp.float32),
    )


def kernel(x, edge_index, A0, A1, ws, W1, b1, W2, b2):
    n_nodes, f_in = x.shape
    n_edges = A0.shape[0]
    h = W1.shape[1]
    c_out = W2.shape[1]

    group = NC * NS * CH * LANES
    n_rows = ((n_edges + group - 1) // group) * (group // LANES)
    e_pad = n_rows * LANES
    pad = e_pad - n_edges

    iot = jnp.arange(pad, dtype=jnp.int32) % n_nodes
    rc2d = jnp.concatenate(
        [edge_index[0], iot, edge_index[1], iot]).reshape(2, n_rows, LANES)
    zpad = jnp.zeros((pad,), jnp.float32)
    a01_2d = jnp.concatenate([A0, zpad, A1, zpad]).reshape(2, n_rows, LANES)
    ws0s = jnp.full((16,), ws[0], jnp.float32)
    ws1s = jnp.full((16,), ws[1], jnp.float32)
    zn = jnp.zeros((n_nodes,), jnp.float32)
    zh = jnp.zeros((n_nodes, h), jnp.float32)
    zc = jnp.zeros((n_nodes, c_out), jnp.float32)

    blk = 1000
    xw1 = _tc_matmul1(n_nodes, f_in, h, blk)(x, W1)

    w2d, degp = _sc_deg_kernel(n_nodes, n_rows)(a01_2d, rc2d, ws0s, ws1s, zn)

    xs1, dis, inv = _tc_norm(n_nodes, h, blk)(
        degp[0].reshape(n_nodes, 1), degp[1].reshape(n_nodes, 1), xw1)

    acc1 = _sc_msg_kernel(n_nodes, n_rows, h)(xs1, rc2d, w2d, zh)

    xw2, xs2 = _tc_layer1_finish(n_nodes, h, c_out, blk)(
        acc1, acc1, xw1, dis, inv, b1.reshape(1, h), W2)

    acc2 = _sc_msg_kernel(n_nodes, n_rows, c_out)(xs2, rc2d, w2d, zc)

    out = _tc_layer2_finish(n_nodes, c_out, blk)(
        acc2, acc2, xw2, dis, inv, b2.reshape(1, c_out))
    return out
